# baseline (speedup 1.0000x reference)
import functools

import jax
import jax.numpy as jnp
from jax import lax
from jax.experimental import pallas as pl
from jax.experimental.pallas import tpu as pltpu
from jax.experimental.pallas import tpu_sc as plsc

_LANES = 16
_G = 16


@functools.cache
def _build(B: int, T: int, V: int, D: int):
    info = plsc.get_sparse_core_info()
    nw = info.num_cores * info.num_subcores
    t_per_w = T // nw
    ng = t_per_w // _G
    mesh = plsc.VectorSubcoreMesh(core_axis_name="c", subcore_axis_name="s")

    scratch = (
        [pltpu.VMEM((B, t_per_w), jnp.int32)]
        + [pltpu.VMEM((_G, D), jnp.float32) for _ in range(2)]
        + [pltpu.VMEM((B, _G, D), jnp.float32) for _ in range(2)]
        + [pltpu.SemaphoreType.DMA for _ in range(2 + 2 * B + 2)]
    )

    @functools.partial(
        pl.kernel,
        mesh=mesh,
        out_type=jax.ShapeDtypeStruct((B, T, D), jnp.float32),
        scratch_types=scratch,
    )
    def emb_kernel(idx_hbm, wte_hbm, wpe_hbm, out_hbm, idx_v, *rest):
        wv = rest[0:2]
        bufs = rest[2:4]
        sems = rest[4:]
        wsem = sems[0:2]
        gsem = (sems[2:2 + B], sems[2 + B:2 + 2 * B])
        ssem = sems[2 + 2 * B:]

        wid = lax.axis_index("s") * info.num_cores + lax.axis_index("c")
        t0 = wid * t_per_w

        for b in range(B):
            pltpu.sync_copy(idx_hbm.at[b, pl.ds(t0, t_per_w)], idx_v.at[b])

        def start_gather(h, b):
            g = h % 2
            return pltpu.async_copy(
                wte_hbm.at[idx_v.at[b, pl.ds(h * _G, _G)]], bufs[g].at[b],
                gsem[g][b])

        gathers = [[None] * B, [None] * B]
        stores = [None, None]
        for b in range(B):
            gathers[0][b] = start_gather(0, b)
        pltpu.sync_copy(wpe_hbm.at[pl.ds(t0, _G)], wv[0])
        wpe_cp = [None, None]
        wpe_cp[1] = pltpu.async_copy(wpe_hbm.at[pl.ds(t0 + _G, _G)], wv[1], wsem[1])

        for h in range(ng):
            g = h % 2
            g2 = (h + 1) % 2
            if h + 1 < ng:
                if stores[g2] is not None:
                    stores[g2].wait()
                    stores[g2] = None
                for b in range(B):
                    gathers[g2][b] = start_gather(h + 1, b)
            if wpe_cp[g] is not None:
                wpe_cp[g].wait()
                wpe_cp[g] = None
            for b in range(B):
                gathers[g][b].wait()
            wvg = wv[g]
            bg = bufs[g]

            def add_row(i, carry):
                for j in range(D // _LANES):
                    sl = pl.ds(j * _LANES, _LANES)
                    w = wvg[i, sl]
                    for b in range(B):
                        bg[b, i, sl] = bg[b, i, sl] + w
                return carry

            lax.fori_loop(0, _G, add_row, 0, unroll=2)
            if h + 2 < ng:
                wpe_cp[g] = pltpu.async_copy(
                    wpe_hbm.at[pl.ds(t0 + (h + 2) * _G, _G)], wvg, wsem[g])
            stores[g] = pltpu.async_copy(
                bg, out_hbm.at[:, pl.ds(t0 + h * _G, _G)], ssem[g])
        for st in stores:
            if st is not None:
                st.wait()

    return emb_kernel


def kernel(idx, wte, wpe):
    b, t = idx.shape
    v, d = wte.shape
    idx32 = idx.astype(jnp.int32)
    return _build(b, t, v, d)(idx32, wte, wpe)


# speedup vs baseline: 1.8849x; 1.8849x over previous
import functools

import jax
import jax.numpy as jnp
from jax import lax
from jax.experimental import pallas as pl
from jax.experimental.pallas import tpu as pltpu
from jax.experimental.pallas import tpu_sc as plsc

_LANES = 16
_G = 16


@functools.cache
def _build(B: int, T: int, V: int, D: int):
    info = plsc.get_sparse_core_info()
    nw = info.num_cores * info.num_subcores
    t_per_w = T // nw
    ng = t_per_w // _G
    mesh = plsc.VectorSubcoreMesh(core_axis_name="c", subcore_axis_name="s")

    scratch = (
        [pltpu.VMEM((B, t_per_w), jnp.int32)]
        + [pltpu.VMEM((_G, D), jnp.float32) for _ in range(2)]
        + [pltpu.VMEM((_G, D), jnp.float32) for _ in range(2 * B)]
        + [pltpu.SemaphoreType.DMA for _ in range(2 + 2 * B + 2 * B)]
    )

    @functools.partial(
        pl.kernel,
        mesh=mesh,
        out_type=jax.ShapeDtypeStruct((B * T, D), jnp.float32),
        scratch_types=scratch,
    )
    def emb_kernel(idx_hbm, wte_hbm, wpe_hbm, out_hbm, idx_v, *rest):
        wv = rest[0:2]
        bufs = (rest[2:2 + B], rest[2 + B:2 + 2 * B])
        sems = rest[2 + 2 * B:]
        wsem = sems[0:2]
        gsem = (sems[2:2 + B], sems[2 + B:2 + 2 * B])
        ssem = (sems[2 + 2 * B:2 + 3 * B], sems[2 + 3 * B:2 + 4 * B])

        wid = lax.axis_index("s") * info.num_cores + lax.axis_index("c")
        t0 = wid * t_per_w

        for b in range(B):
            pltpu.sync_copy(idx_hbm.at[pl.ds(b * T + t0, t_per_w)], idx_v.at[b])

        def start_gather(h, b):
            g = h % 2
            return pltpu.async_copy(
                wte_hbm.at[idx_v.at[b, pl.ds(h * _G, _G)]], bufs[g][b], gsem[g][b])

        gathers = [[None] * B, [None] * B]
        stores = [[None] * B, [None] * B]
        for b in range(B):
            gathers[0][b] = start_gather(0, b)
        pltpu.sync_copy(wpe_hbm.at[pl.ds(t0, _G)], wv[0])
        wpe_cp = [None, None]
        wpe_cp[1] = pltpu.async_copy(wpe_hbm.at[pl.ds(t0 + _G, _G)], wv[1], wsem[1])

        for h in range(ng):
            g = h % 2
            g2 = (h + 1) % 2
            if h + 1 < ng:
                for b in range(B):
                    if stores[g2][b] is not None:
                        stores[g2][b].wait()
                        stores[g2][b] = None
                    gathers[g2][b] = start_gather(h + 1, b)
            if wpe_cp[g] is not None:
                wpe_cp[g].wait()
                wpe_cp[g] = None
            for b in range(B):
                gathers[g][b].wait()
            wvg = wv[g]
            bg = bufs[g]

            def add_row(i, carry):
                for j in range(D // _LANES):
                    sl = pl.ds(j * _LANES, _LANES)
                    w = wvg[i, sl]
                    for b in range(B):
                        bg[b][i, sl] = bg[b][i, sl] + w
                return carry

            lax.fori_loop(0, _G, add_row, 0, unroll=4)
            if h + 2 < ng:
                wpe_cp[g] = pltpu.async_copy(
                    wpe_hbm.at[pl.ds(t0 + (h + 2) * _G, _G)], wvg, wsem[g])
            for b in range(B):
                stores[g][b] = pltpu.async_copy(
                    bg[b], out_hbm.at[pl.ds(b * T + t0 + h * _G, _G)], ssem[g][b])
        for side in stores:
            for st in side:
                if st is not None:
                    st.wait()

    return emb_kernel


def kernel(idx, wte, wpe):
    b, t = idx.shape
    v, d = wte.shape
    idx_flat = idx.reshape(b * t).astype(jnp.int32)
    out = _build(b, t, v, d)(idx_flat, wte, wpe)
    return out.reshape(b, t, d)


# speedup vs baseline: 2.0893x; 1.1084x over previous
import functools

import jax
import jax.numpy as jnp
from jax import lax
from jax.experimental import pallas as pl
from jax.experimental.pallas import tpu as pltpu
from jax.experimental.pallas import tpu_sc as plsc

_LANES = 16
_G = 16


@functools.cache
def _build(B: int, T: int, V: int, D: int):
    info = plsc.get_sparse_core_info()
    nw = info.num_cores * info.num_subcores
    t_per_w = T // nw
    ng = t_per_w // _G
    mesh = plsc.VectorSubcoreMesh(core_axis_name="c", subcore_axis_name="s")

    scratch = (
        [pltpu.VMEM((B, t_per_w), jnp.int32)]
        + [pltpu.VMEM((_G, D), jnp.float32) for _ in range(2)]
        + [pltpu.VMEM((_G, D), jnp.float32) for _ in range(2 * B)]
        + [pltpu.SemaphoreType.DMA for _ in range(2 + 2 * B + 2 * B)]
    )

    @functools.partial(
        pl.kernel,
        mesh=mesh,
        out_type=jax.ShapeDtypeStruct((B * T, D), jnp.float32),
        scratch_types=scratch,
    )
    def emb_kernel(idx_hbm, wte_hbm, wpe_hbm, out_hbm, idx_v, *rest):
        wv = rest[0:2]
        bufs = (rest[2:2 + B], rest[2 + B:2 + 2 * B])
        sems = rest[2 + 2 * B:]
        wsem = sems[0:2]
        gsem = (sems[2:2 + B], sems[2 + B:2 + 2 * B])
        ssem = (sems[2 + 2 * B:2 + 3 * B], sems[2 + 3 * B:2 + 4 * B])

        wid = lax.axis_index("s") * info.num_cores + lax.axis_index("c")
        t0 = wid * t_per_w

        for b in range(B):
            pltpu.sync_copy(idx_hbm.at[pl.ds(b * T + t0, t_per_w)], idx_v.at[b])

        def start_gather(h, b):
            g = h % 2
            return pltpu.async_copy(
                wte_hbm.at[idx_v.at[b, pl.ds(h * _G, _G)]], bufs[g][b], gsem[g][b])

        gathers = [[None] * B, [None] * B]
        stores = [[None] * B, [None] * B]
        for b in range(B):
            gathers[0][b] = start_gather(0, b)
        pltpu.sync_copy(wpe_hbm.at[pl.ds(t0, _G)], wv[0])
        wpe_cp = [None, None]
        wpe_cp[1] = pltpu.async_copy(wpe_hbm.at[pl.ds(t0 + _G, _G)], wv[1], wsem[1])

        for h in range(ng):
            g = h % 2
            g2 = (h + 1) % 2
            if h + 1 < ng:
                for b in range(B):
                    if stores[g2][b] is not None:
                        stores[g2][b].wait()
                        stores[g2][b] = None
                    gathers[g2][b] = start_gather(h + 1, b)
            if wpe_cp[g] is not None:
                wpe_cp[g].wait()
                wpe_cp[g] = None
            for b in range(B):
                gathers[g][b].wait()
            wvg = wv[g]
            bg = bufs[g]

            @plsc.parallel_loop(0, _G, unroll=2)
            def _add(i):
                for j in range(D // _LANES):
                    sl = pl.ds(j * _LANES, _LANES)
                    w = wvg[i, sl]
                    for b in range(B):
                        bg[b][i, sl] = bg[b][i, sl] + w
            if h + 2 < ng:
                wpe_cp[g] = pltpu.async_copy(
                    wpe_hbm.at[pl.ds(t0 + (h + 2) * _G, _G)], wvg, wsem[g])
            for b in range(B):
                stores[g][b] = pltpu.async_copy(
                    bg[b], out_hbm.at[pl.ds(b * T + t0 + h * _G, _G)], ssem[g][b])
        for side in stores:
            for st in side:
                if st is not None:
                    st.wait()

    return emb_kernel


def kernel(idx, wte, wpe):
    b, t = idx.shape
    v, d = wte.shape
    idx_flat = idx.reshape(b * t).astype(jnp.int32)
    out = _build(b, t, v, d)(idx_flat, wte, wpe)
    return out.reshape(b, t, d)
